# Initial kernel scaffold; baseline (speedup 1.0000x reference)
import jax
import jax.numpy as jnp
from jax.experimental import pallas as pl


def kernel(embeddings, labels, graph_ids, categories):
    raise NotImplementedError("write your pallas kernel here")


# speedup vs baseline: 74.9386x; 74.9386x over previous
import functools

import jax
import jax.numpy as jnp
from jax import lax
from jax.experimental import pallas as pl
from jax.experimental.pallas import tpu as pltpu
from jax.experimental.pallas import tpu_sc as plsc

_N = 50000
_D = 128
_NLAB = 2000
_NGRAPH = 8
_LAB_PER_SC = _NLAB // 2
_ROWS = _LAB_PER_SC * _NGRAPH
_NDUMMY = 8
_TROWS = _ROWS + _NDUMMY
_NBUCKET = _NLAB * _NGRAPH
_CT = 8192
_CHUNK = 128
_NFULL = _N // _CHUNK
_TAIL = _N - _NFULL * _CHUNK
_PAIRS_PER_TILE = 32
_MAXSAMP = 10000
_NEG_GROUPS = _MAXSAMP // 16


def _norm_pack_body(e_ref, lab_ref, gid_ref, cat_ref, en_ref, pk_ref):
    e = e_ref[...]
    nrm = jnp.clip(jnp.sqrt(jnp.sum(e * e, axis=1, keepdims=True)), 1e-12)
    en_ref[...] = e / nrm
    lab = lab_ref[...]
    gid = gid_ref[...]
    cat = cat_ref[...]
    pk_ref[...] = lab * 16 + gid * 2 + (cat < 3).astype(jnp.int32)


def _norm_pack(emb, labels, gids, cats):
    nb = 25
    blk = _N // nb
    lab3 = labels.reshape(nb, 1, blk)
    gid3 = gids.reshape(nb, 1, blk)
    cat3 = cats.reshape(nb, 1, blk)
    en, pk3 = pl.pallas_call(
        _norm_pack_body,
        grid=(nb,),
        in_specs=[
            pl.BlockSpec((blk, _D), lambda i: (i, 0)),
            pl.BlockSpec((1, 1, blk), lambda i: (i, 0, 0)),
            pl.BlockSpec((1, 1, blk), lambda i: (i, 0, 0)),
            pl.BlockSpec((1, 1, blk), lambda i: (i, 0, 0)),
        ],
        out_specs=[
            pl.BlockSpec((blk, _D), lambda i: (i, 0)),
            pl.BlockSpec((1, 1, blk), lambda i: (i, 0, 0)),
        ],
        out_shape=[
            jax.ShapeDtypeStruct((_N, _D), jnp.float32),
            jax.ShapeDtypeStruct((nb, 1, blk), jnp.int32),
        ],
    )(emb, lab3, gid3, cat3)
    return en, pk3.reshape(_N)


_MESH = plsc.VectorSubcoreMesh(core_axis_name="c", subcore_axis_name="s")


def _bucket_idx(p, core, lane8):
    lab = p >> 4
    gid = (p >> 1) & 7
    cons = p & 1
    mylab = lab - core * _LAB_PER_SC
    mine = (cons == 1) & (mylab >= 0) & (mylab < _LAB_PER_SC)
    return jnp.where(mine, mylab * _NGRAPH + gid, _ROWS + lane8)


@functools.partial(
    pl.kernel,
    out_type=[
        jax.ShapeDtypeStruct((32, 32), jnp.float32),
        jax.ShapeDtypeStruct((32, _CT), jnp.float32),
    ],
    mesh=_MESH,
    compiler_params=pltpu.CompilerParams(needs_layout_passes=False),
    scratch_types=[
        pltpu.VMEM_SHARED((_TROWS, _D), jnp.float32),
        pltpu.VMEM((_CHUNK, _D), jnp.float32),
        pltpu.VMEM((_TAIL, _D), jnp.float32),
        pltpu.VMEM((_CT,), jnp.float32),
        pltpu.VMEM((_CHUNK,), jnp.int32),
        pltpu.VMEM((_TAIL,), jnp.int32),
        pltpu.VMEM((_CHUNK,), jnp.int32),
        pltpu.VMEM((_TAIL,), jnp.int32),
        pltpu.VMEM((8, _D), jnp.float32),
        pltpu.VMEM((16, _D), jnp.float32),
        pltpu.VMEM((2, 16), jnp.float32),
        pltpu.VMEM((32,), jnp.float32),
    ],
)
def _bucket_kernel(en_hbm, pk_hbm, out_hbm, cnt_hbm,
                   table, ebuf, ebuf_t, ctloc,
                   pbuf, pbuf_t, ibuf, ibuf_t,
                   zbuf, lbuf, abuf, obuf):
    c = lax.axis_index("c")
    s = lax.axis_index("s")
    lane = lax.iota(jnp.int32, 16)
    lane8 = lane & 7
    zv = jnp.zeros((16,), jnp.float32)
    onev = jnp.ones((16,), jnp.float32)

    for r in range(8):
        for q in range(_D // 16):
            zbuf[r, pl.ds(q * 16, 16)] = zv
    abuf[0, pl.ds(0, 16)] = zv
    abuf[1, pl.ds(0, 16)] = zv

    def _zero_ct(k, _):
        ctloc[pl.ds(k * 16, 16)] = zv
        return 0

    lax.fori_loop(0, _CT // 16, _zero_ct, 0)

    ngroups = _TROWS // 8

    def _zero_group(k, _):
        grp = k * 16 + s

        @pl.when(grp < ngroups)
        def _zt():
            pltpu.sync_copy(zbuf, table.at[pl.ds(grp * 8, 8), :])

        return 0

    lax.fori_loop(0, (ngroups + 15) // 16, _zero_group, 0)

    plsc.subcore_barrier()

    def _chunk_body(k, _):
        ch = k * 16 + s

        @pl.when(ch < _NFULL)
        def _full_chunk():
            n0 = ch * _CHUNK
            pltpu.sync_copy(pk_hbm.at[pl.ds(n0, _CHUNK)], pbuf)
            pltpu.sync_copy(en_hbm.at[pl.ds(n0, _CHUNK), :], ebuf)
            for j in range(_CHUNK // 16):
                p = pbuf[pl.ds(j * 16, 16)]
                idx = _bucket_idx(p, c, lane8)
                ibuf[pl.ds(j * 16, 16)] = idx
                plsc.addupdate_scatter(ctloc, [idx], onev)
            pltpu.sync_copy(ebuf, table.at[ibuf], add=True)

        return 0

    lax.fori_loop(0, 25, _chunk_body, 0)

    @pl.when(s == 6)
    def _tail_chunk():
        n0 = _NFULL * _CHUNK
        pltpu.sync_copy(pk_hbm.at[pl.ds(n0, _TAIL)], pbuf_t)
        pltpu.sync_copy(en_hbm.at[pl.ds(n0, _TAIL), :], ebuf_t)
        for j in range(_TAIL // 16):
            p = pbuf_t[pl.ds(j * 16, 16)]
            idx = _bucket_idx(p, c, lane8)
            ibuf_t[pl.ds(j * 16, 16)] = idx
            plsc.addupdate_scatter(ctloc, [idx], onev)
        pltpu.sync_copy(ebuf_t, table.at[ibuf_t], add=True)

    pltpu.sync_copy(ctloc, cnt_hbm.at[c * 16 + s])

    def _pair_body(j, _):
        lp = s * _PAIRS_PER_TILE + j

        @pl.when(lp < _LAB_PER_SC // 2)
        def _one_pair():
            pltpu.sync_copy(table.at[pl.ds(lp * 16, 16), :], lbuf)
            acc_a = jnp.zeros((16,), jnp.float32)
            acc_b = jnp.zeros((16,), jnp.float32)
            for half in range(2):
                svec = [jnp.zeros((16,), jnp.float32) for _ in range(_D // 16)]
                for g in range(_NGRAPH):
                    for q in range(_D // 16):
                        v = lbuf[half * 8 + g, pl.ds(q * 16, 16)]
                        acc_a = acc_a + v * v
                        svec[q] = svec[q] + v
                for q in range(_D // 16):
                    acc_b = acc_b + svec[q] * svec[q]
            abuf[0, pl.ds(0, 16)] = abuf[0, pl.ds(0, 16)] + acc_a
            abuf[1, pl.ds(0, 16)] = abuf[1, pl.ds(0, 16)] + acc_b

        return 0

    lax.fori_loop(0, _PAIRS_PER_TILE, _pair_body, 0)

    obuf[pl.ds(0, 16)] = abuf[0, pl.ds(0, 16)]
    obuf[pl.ds(16, 16)] = abuf[1, pl.ds(0, 16)]
    pltpu.sync_copy(obuf, out_hbm.at[c * 16 + s])


@functools.partial(
    pl.kernel,
    out_type=jax.ShapeDtypeStruct((32, 32), jnp.float32),
    mesh=_MESH,
    compiler_params=pltpu.CompilerParams(needs_layout_passes=False),
    scratch_types=[
        pltpu.VMEM((_N,), jnp.int32),
        pltpu.VMEM((_MAXSAMP,), jnp.int32),
        pltpu.VMEM((_MAXSAMP,), jnp.int32),
        pltpu.VMEM((16,), jnp.int32),
        pltpu.VMEM((16,), jnp.int32),
        pltpu.VMEM((16,), jnp.int32),
        pltpu.VMEM((16, _D), jnp.float32),
        pltpu.VMEM((16, _D), jnp.float32),
        pltpu.VMEM((256,), jnp.float32),
        pltpu.VMEM((32,), jnp.float32),
        pltpu.SemaphoreType.DMA,
    ],
)
def _neg_kernel(en_hbm, pk_hbm, i1_hbm, i2_hbm, ns_hbm, out_hbm,
                pkbuf, i1buf, i2buf, nsbuf, g1buf, g2buf,
                rows1, rows2, tbuf, obuf, sem):
    c = lax.axis_index("c")
    s = lax.axis_index("s")
    wid = c * 16 + s
    lane = lax.iota(jnp.int32, 16)
    pltpu.sync_copy(pk_hbm, pkbuf)
    pltpu.sync_copy(i1_hbm, i1buf)
    pltpu.sync_copy(i2_hbm, i2buf)
    pltpu.sync_copy(ns_hbm, nsbuf)
    nsamp = nsbuf[pl.ds(0, 16)][0]
    obuf[pl.ds(0, 16)] = jnp.zeros((16,), jnp.float32)
    obuf[pl.ds(16, 16)] = jnp.zeros((16,), jnp.float32)

    def _group_body(k, _):
        g = k * 32 + wid

        @pl.when(g < _NEG_GROUPS)
        def _group():
            base = g * 16
            i1 = i1buf[pl.ds(base, 16)]
            i2 = i2buf[pl.ds(base, 16)]
            p1 = plsc.load_gather(pkbuf, [i1])
            p2 = plsc.load_gather(pkbuf, [i2])
            valid = (((p1 >> 4) != (p2 >> 4))
                     & (((p1 >> 1) & 7) != ((p2 >> 1) & 7))
                     & (((p1 | p2) & 1) == 1)
                     & ((base + lane) < nsamp))
            validf = jnp.where(valid, 1.0, 0.0).astype(jnp.float32)
            g1buf[pl.ds(0, 16)] = i1
            g2buf[pl.ds(0, 16)] = i2
            pltpu.async_copy(en_hbm.at[g1buf], rows1, sem).wait()
            pltpu.async_copy(en_hbm.at[g2buf], rows2, sem).wait()
            for j in range(16):
                acc = rows1[j, pl.ds(0, 16)] * rows2[j, pl.ds(0, 16)]
                for q in range(1, _D // 16):
                    acc = acc + (rows1[j, pl.ds(q * 16, 16)]
                                 * rows2[j, pl.ds(q * 16, 16)])
                tbuf[pl.ds(j * 16, 16)] = acc
            dots = plsc.load_gather(tbuf, [lane * 16])
            for q in range(1, 16):
                dots = dots + plsc.load_gather(tbuf, [lane * 16 + q])
            dv = jnp.maximum(dots, 0.0) * validf
            obuf[pl.ds(0, 16)] = obuf[pl.ds(0, 16)] + dv
            obuf[pl.ds(16, 16)] = obuf[pl.ds(16, 16)] + validf

        return 0

    lax.fori_loop(0, 20, _group_body, 0)
    pltpu.sync_copy(obuf, out_hbm.at[wid])


def _neg_indices(n_pairs, n_total):
    nk = jax.random.key(123)
    nk1, nk2 = jax.random.split(nk)
    bidx = jnp.clip(n_pairs, 0, 200)
    n_samples = jnp.where(bidx < 200, bidx * 50, _MAXSAMP)

    def make_branch(sz):
        def branch(keys):
            ka, kb = keys
            i1 = jax.random.randint(ka, (sz,), 0, n_total)
            i2 = jax.random.randint(kb, (sz,), 0, n_total)
            return (jnp.pad(i1, (0, _MAXSAMP - sz)),
                    jnp.pad(i2, (0, _MAXSAMP - sz)))
        return branch

    branches = [make_branch(50 * t) for t in range(200)] + [make_branch(_MAXSAMP)]
    idx1, idx2 = lax.switch(bidx, branches, (nk1, nk2))
    return idx1.astype(jnp.int32), idx2.astype(jnp.int32), n_samples


def kernel(embeddings, labels, graph_ids, categories):
    en, packed = _norm_pack(embeddings, labels.astype(jnp.int32),
                            graph_ids.astype(jnp.int32),
                            categories.astype(jnp.int32))
    parts, cnts = _bucket_kernel(en, packed)
    a = jnp.sum(parts[:, :16])
    b = jnp.sum(parts[:, 16:])
    csum = cnts.reshape(2, 16, _CT).sum(axis=1)
    n_lg = csum[:, :_ROWS].reshape(_NLAB, _NGRAPH)
    cq = jnp.sum(n_lg * n_lg)
    n_l = jnp.sum(n_lg, axis=-1)
    dq = jnp.sum(n_l * n_l)
    n_pairs_f = 0.5 * (dq - cq)
    pos_sim = 0.5 * (b - a)
    pos_sum = n_pairs_f - pos_sim
    n_pairs = jnp.round(n_pairs_f).astype(jnp.int32)
    pos_loss = pos_sum / jnp.maximum(n_pairs, 1).astype(jnp.float32)

    idx1, idx2, n_samples = _neg_indices(n_pairs, _N)
    ns_arr = jnp.full((16,), n_samples, dtype=jnp.int32)
    negparts = _neg_kernel(en, packed, idx1, idx2, ns_arr)
    nsum = jnp.sum(negparts[:, :16])
    n_valid = jnp.round(jnp.sum(negparts[:, 16:])).astype(jnp.int32)
    neg_loss = nsum / jnp.maximum(n_valid, 1).astype(jnp.float32)
    total = jnp.where(n_valid > 0, pos_loss + neg_loss, pos_loss)
    return jnp.where(n_pairs > 0, total, jnp.float32(0.0))


# speedup vs baseline: 93.3159x; 1.2452x over previous
import functools

import jax
import jax.numpy as jnp
from jax import lax
from jax.experimental import pallas as pl
from jax.experimental.pallas import tpu as pltpu
from jax.experimental.pallas import tpu_sc as plsc

_N = 50000
_D = 128
_NLAB = 2000
_NGRAPH = 8
_LAB_PER_SC = _NLAB // 2
_ROWS = _LAB_PER_SC * _NGRAPH
_NDUMMY = 8
_TROWS = _ROWS + _NDUMMY
_NBUCKET = _NLAB * _NGRAPH
_CT = 8192
_CHUNK = 128
_NFULL = _N // _CHUNK
_TAIL = _N - _NFULL * _CHUNK
_PAIRS_PER_TILE = 32
_MAXSAMP = 10000
_NEG_GROUPS = _MAXSAMP // 16


def _norm_pack_body(e_ref, lab_ref, gid_ref, cat_ref, en_ref, pk_ref):
    e = e_ref[...]
    nrm = jnp.clip(jnp.sqrt(jnp.sum(e * e, axis=1, keepdims=True)), 1e-12)
    en_ref[...] = e / nrm
    lab = lab_ref[...]
    gid = gid_ref[...]
    cat = cat_ref[...]
    pk_ref[...] = lab * 16 + gid * 2 + (cat < 3).astype(jnp.int32)


def _norm_pack(emb, labels, gids, cats):
    nb = 25
    blk = _N // nb
    lab3 = labels.reshape(nb, 1, blk)
    gid3 = gids.reshape(nb, 1, blk)
    cat3 = cats.reshape(nb, 1, blk)
    en, pk3 = pl.pallas_call(
        _norm_pack_body,
        grid=(nb,),
        in_specs=[
            pl.BlockSpec((blk, _D), lambda i: (i, 0)),
            pl.BlockSpec((1, 1, blk), lambda i: (i, 0, 0)),
            pl.BlockSpec((1, 1, blk), lambda i: (i, 0, 0)),
            pl.BlockSpec((1, 1, blk), lambda i: (i, 0, 0)),
        ],
        out_specs=[
            pl.BlockSpec((blk, _D), lambda i: (i, 0)),
            pl.BlockSpec((1, 1, blk), lambda i: (i, 0, 0)),
        ],
        out_shape=[
            jax.ShapeDtypeStruct((_N, _D), jnp.float32),
            jax.ShapeDtypeStruct((nb, 1, blk), jnp.int32),
        ],
    )(emb, lab3, gid3, cat3)
    return en, pk3.reshape(_N)


_MESH = plsc.VectorSubcoreMesh(core_axis_name="c", subcore_axis_name="s")


def _bucket_idx(p, core, lane8):
    lab = p >> 4
    gid = (p >> 1) & 7
    cons = p & 1
    mylab = lab - core * _LAB_PER_SC
    mine = (cons == 1) & (mylab >= 0) & (mylab < _LAB_PER_SC)
    return jnp.where(mine, mylab * _NGRAPH + gid, _ROWS + lane8)


@functools.partial(
    pl.kernel,
    out_type=[
        jax.ShapeDtypeStruct((32, 32), jnp.float32),
        jax.ShapeDtypeStruct((32, _CT), jnp.float32),
    ],
    mesh=_MESH,
    compiler_params=pltpu.CompilerParams(needs_layout_passes=False),
    scratch_types=[
        pltpu.VMEM_SHARED((_TROWS, _D), jnp.float32),
        pltpu.VMEM((_CHUNK, _D), jnp.float32),
        pltpu.VMEM((_TAIL, _D), jnp.float32),
        pltpu.VMEM((_CT,), jnp.float32),
        pltpu.VMEM((_CHUNK,), jnp.int32),
        pltpu.VMEM((_TAIL,), jnp.int32),
        pltpu.VMEM((_CHUNK,), jnp.int32),
        pltpu.VMEM((_TAIL,), jnp.int32),
        pltpu.VMEM((8, _D), jnp.float32),
        pltpu.VMEM((16, _D), jnp.float32),
        pltpu.VMEM((2, 16), jnp.float32),
        pltpu.VMEM((32,), jnp.float32),
    ],
)
def _bucket_kernel(en_hbm, pk_hbm, out_hbm, cnt_hbm,
                   table, ebuf, ebuf_t, ctloc,
                   pbuf, pbuf_t, ibuf, ibuf_t,
                   zbuf, lbuf, abuf, obuf):
    c = lax.axis_index("c")
    s = lax.axis_index("s")
    lane = lax.iota(jnp.int32, 16)
    lane8 = lane & 7
    zv = jnp.zeros((16,), jnp.float32)
    onev = jnp.ones((16,), jnp.float32)

    for r in range(8):
        for q in range(_D // 16):
            zbuf[r, pl.ds(q * 16, 16)] = zv
    abuf[0, pl.ds(0, 16)] = zv
    abuf[1, pl.ds(0, 16)] = zv

    def _zero_ct(k, _):
        ctloc[pl.ds(k * 16, 16)] = zv
        return 0

    lax.fori_loop(0, _CT // 16, _zero_ct, 0)

    ngroups = _TROWS // 8

    def _zero_group(k, _):
        grp = k * 16 + s

        @pl.when(grp < ngroups)
        def _zt():
            pltpu.sync_copy(zbuf, table.at[pl.ds(grp * 8, 8), :])

        return 0

    lax.fori_loop(0, (ngroups + 15) // 16, _zero_group, 0)

    plsc.subcore_barrier()

    def _chunk_body(k, _):
        ch = k * 16 + s

        @pl.when(ch < _NFULL)
        def _full_chunk():
            n0 = ch * _CHUNK
            pltpu.sync_copy(pk_hbm.at[pl.ds(n0, _CHUNK)], pbuf)
            pltpu.sync_copy(en_hbm.at[pl.ds(n0, _CHUNK), :], ebuf)
            for j in range(_CHUNK // 16):
                p = pbuf[pl.ds(j * 16, 16)]
                idx = _bucket_idx(p, c, lane8)
                ibuf[pl.ds(j * 16, 16)] = idx
                plsc.addupdate_scatter(ctloc, [idx], onev)
            pltpu.sync_copy(ebuf, table.at[ibuf], add=True)

        return 0

    lax.fori_loop(0, 25, _chunk_body, 0)

    @pl.when(s == 6)
    def _tail_chunk():
        n0 = _NFULL * _CHUNK
        pltpu.sync_copy(pk_hbm.at[pl.ds(n0, _TAIL)], pbuf_t)
        pltpu.sync_copy(en_hbm.at[pl.ds(n0, _TAIL), :], ebuf_t)
        for j in range(_TAIL // 16):
            p = pbuf_t[pl.ds(j * 16, 16)]
            idx = _bucket_idx(p, c, lane8)
            ibuf_t[pl.ds(j * 16, 16)] = idx
            plsc.addupdate_scatter(ctloc, [idx], onev)
        pltpu.sync_copy(ebuf_t, table.at[ibuf_t], add=True)

    pltpu.sync_copy(ctloc, cnt_hbm.at[c * 16 + s])

    def _pair_body(j, _):
        lp = s * _PAIRS_PER_TILE + j

        @pl.when(lp < _LAB_PER_SC // 2)
        def _one_pair():
            pltpu.sync_copy(table.at[pl.ds(lp * 16, 16), :], lbuf)
            acc_a = jnp.zeros((16,), jnp.float32)
            acc_b = jnp.zeros((16,), jnp.float32)
            for half in range(2):
                svec = [jnp.zeros((16,), jnp.float32) for _ in range(_D // 16)]
                for g in range(_NGRAPH):
                    for q in range(_D // 16):
                        v = lbuf[half * 8 + g, pl.ds(q * 16, 16)]
                        acc_a = acc_a + v * v
                        svec[q] = svec[q] + v
                for q in range(_D // 16):
                    acc_b = acc_b + svec[q] * svec[q]
            abuf[0, pl.ds(0, 16)] = abuf[0, pl.ds(0, 16)] + acc_a
            abuf[1, pl.ds(0, 16)] = abuf[1, pl.ds(0, 16)] + acc_b

        return 0

    lax.fori_loop(0, _PAIRS_PER_TILE, _pair_body, 0)

    obuf[pl.ds(0, 16)] = abuf[0, pl.ds(0, 16)]
    obuf[pl.ds(16, 16)] = abuf[1, pl.ds(0, 16)]
    pltpu.sync_copy(obuf, out_hbm.at[c * 16 + s])


@functools.partial(
    pl.kernel,
    out_type=jax.ShapeDtypeStruct((32, 32), jnp.float32),
    mesh=_MESH,
    compiler_params=pltpu.CompilerParams(needs_layout_passes=False),
    scratch_types=[
        pltpu.VMEM((_N,), jnp.int32),
        pltpu.VMEM((_MAXSAMP,), jnp.int32),
        pltpu.VMEM((_MAXSAMP,), jnp.int32),
        pltpu.VMEM((16,), jnp.int32),
        pltpu.VMEM((16,), jnp.int32),
        pltpu.VMEM((16,), jnp.int32),
        pltpu.VMEM((16, _D), jnp.float32),
        pltpu.VMEM((16, _D), jnp.float32),
        pltpu.VMEM((256,), jnp.float32),
        pltpu.VMEM((32,), jnp.float32),
        pltpu.SemaphoreType.DMA,
    ],
)
def _neg_kernel(en_hbm, pk_hbm, i1_hbm, i2_hbm, ns_hbm, out_hbm,
                pkbuf, i1buf, i2buf, nsbuf, g1buf, g2buf,
                rows1, rows2, tbuf, obuf, sem):
    c = lax.axis_index("c")
    s = lax.axis_index("s")
    wid = c * 16 + s
    lane = lax.iota(jnp.int32, 16)
    pltpu.sync_copy(pk_hbm, pkbuf)
    pltpu.sync_copy(i1_hbm, i1buf)
    pltpu.sync_copy(i2_hbm, i2buf)
    pltpu.sync_copy(ns_hbm, nsbuf)
    nsamp = nsbuf[pl.ds(0, 16)][0]
    obuf[pl.ds(0, 16)] = jnp.zeros((16,), jnp.float32)
    obuf[pl.ds(16, 16)] = jnp.zeros((16,), jnp.float32)

    def _group_body(k, _):
        g = k * 32 + wid

        @pl.when(g < _NEG_GROUPS)
        def _group():
            base = g * 16
            i1 = i1buf[pl.ds(base, 16)]
            i2 = i2buf[pl.ds(base, 16)]
            p1 = plsc.load_gather(pkbuf, [i1])
            p2 = plsc.load_gather(pkbuf, [i2])
            valid = (((p1 >> 4) != (p2 >> 4))
                     & (((p1 >> 1) & 7) != ((p2 >> 1) & 7))
                     & (((p1 | p2) & 1) == 1)
                     & ((base + lane) < nsamp))
            validf = jnp.where(valid, 1.0, 0.0).astype(jnp.float32)
            g1buf[pl.ds(0, 16)] = i1
            g2buf[pl.ds(0, 16)] = i2
            pltpu.async_copy(en_hbm.at[g1buf], rows1, sem).wait()
            pltpu.async_copy(en_hbm.at[g2buf], rows2, sem).wait()
            for j in range(16):
                acc = rows1[j, pl.ds(0, 16)] * rows2[j, pl.ds(0, 16)]
                for q in range(1, _D // 16):
                    acc = acc + (rows1[j, pl.ds(q * 16, 16)]
                                 * rows2[j, pl.ds(q * 16, 16)])
                tbuf[pl.ds(j * 16, 16)] = acc
            dots = plsc.load_gather(tbuf, [lane * 16])
            for q in range(1, 16):
                dots = dots + plsc.load_gather(tbuf, [lane * 16 + q])
            dv = jnp.maximum(dots, 0.0) * validf
            obuf[pl.ds(0, 16)] = obuf[pl.ds(0, 16)] + dv
            obuf[pl.ds(16, 16)] = obuf[pl.ds(16, 16)] + validf

        return 0

    lax.fori_loop(0, 20, _group_body, 0)
    pltpu.sync_copy(obuf, out_hbm.at[wid])


def _neg_indices(n_pairs, n_total):
    nk = jax.random.key(123)
    nk1, nk2 = jax.random.split(nk)
    bidx = jnp.clip(n_pairs, 0, 200)
    n_samples = jnp.where(bidx < 200, bidx * 50, _MAXSAMP)

    def make_branch(sz):
        def branch(keys):
            ka, kb = keys
            i1 = jax.random.randint(ka, (sz,), 0, n_total)
            i2 = jax.random.randint(kb, (sz,), 0, n_total)
            return (jnp.pad(i1, (0, _MAXSAMP - sz)),
                    jnp.pad(i2, (0, _MAXSAMP - sz)))
        return branch

    branches = [make_branch(50 * t) for t in range(200)] + [make_branch(_MAXSAMP)]
    idx1, idx2 = lax.switch(bidx, branches, (nk1, nk2))
    return idx1.astype(jnp.int32), idx2.astype(jnp.int32), n_samples


def kernel(embeddings, labels, graph_ids, categories):
    en, packed = _norm_pack(embeddings, labels.astype(jnp.int32),
                            graph_ids.astype(jnp.int32),
                            categories.astype(jnp.int32))
    nk = jax.random.key(123)
    nk1, nk2 = jax.random.split(nk)
    idx1_c = jax.random.randint(nk1, (_MAXSAMP,), 0, _N).astype(jnp.int32)
    idx2_c = jax.random.randint(nk2, (_MAXSAMP,), 0, _N).astype(jnp.int32)
    ns_full = jnp.full((16,), _MAXSAMP, dtype=jnp.int32)
    negparts_spec = _neg_kernel(en, packed, idx1_c, idx2_c, ns_full)
    parts, cnts = _bucket_kernel(en, packed)
    a = jnp.sum(parts[:, :16])
    b = jnp.sum(parts[:, 16:])
    csum = cnts.reshape(2, 16, _CT).sum(axis=1)
    n_lg = csum[:, :_ROWS].reshape(_NLAB, _NGRAPH)
    cq = jnp.sum(n_lg * n_lg)
    n_l = jnp.sum(n_lg, axis=-1)
    dq = jnp.sum(n_l * n_l)
    n_pairs_f = 0.5 * (dq - cq)
    pos_sim = 0.5 * (b - a)
    pos_sum = n_pairs_f - pos_sim
    n_pairs = jnp.round(n_pairs_f).astype(jnp.int32)
    pos_loss = pos_sum / jnp.maximum(n_pairs, 1).astype(jnp.float32)

    def _rare_path(_):
        idx1, idx2, n_samples = _neg_indices(n_pairs, _N)
        ns_arr = jnp.full((16,), n_samples, dtype=jnp.int32)
        return _neg_kernel(en, packed, idx1, idx2, ns_arr)

    negparts = lax.cond(n_pairs >= 200,
                        lambda _: negparts_spec, _rare_path, 0)
    nsum = jnp.sum(negparts[:, :16])
    n_valid = jnp.round(jnp.sum(negparts[:, 16:])).astype(jnp.int32)
    neg_loss = nsum / jnp.maximum(n_valid, 1).astype(jnp.float32)
    total = jnp.where(n_valid > 0, pos_loss + neg_loss, pos_loss)
    return jnp.where(n_pairs > 0, total, jnp.float32(0.0))


# speedup vs baseline: 98.7965x; 1.0587x over previous
import functools

import jax
import jax.numpy as jnp
from jax import lax
from jax.experimental import pallas as pl
from jax.experimental.pallas import tpu as pltpu
from jax.experimental.pallas import tpu_sc as plsc

_N = 50000
_D = 128
_NLAB = 2000
_NGRAPH = 8
_LAB_PER_SC = _NLAB // 2
_ROWS = _LAB_PER_SC * _NGRAPH
_NDUMMY = 8
_TROWS = _ROWS + _NDUMMY
_NBUCKET = _NLAB * _NGRAPH
_CT = 8192
_CHUNK = 128
_NFULL = _N // _CHUNK
_TAIL = _N - _NFULL * _CHUNK
_PAIRS_PER_TILE = 32
_MAXSAMP = 10000
_NEG_GROUPS = _MAXSAMP // 16


def _norm_pack_body(e_ref, lab_ref, gid_ref, cat_ref, en_ref, pk_ref):
    e = e_ref[...]
    nrm = jnp.clip(jnp.sqrt(jnp.sum(e * e, axis=1, keepdims=True)), 1e-12)
    en_ref[...] = e / nrm
    lab = lab_ref[...]
    gid = gid_ref[...]
    cat = cat_ref[...]
    pk_ref[...] = lab * 16 + gid * 2 + (cat < 3).astype(jnp.int32)


def _norm_pack(emb, labels, gids, cats):
    nb = 25
    blk = _N // nb
    lab3 = labels.reshape(nb, 1, blk)
    gid3 = gids.reshape(nb, 1, blk)
    cat3 = cats.reshape(nb, 1, blk)
    en, pk3 = pl.pallas_call(
        _norm_pack_body,
        grid=(nb,),
        in_specs=[
            pl.BlockSpec((blk, _D), lambda i: (i, 0)),
            pl.BlockSpec((1, 1, blk), lambda i: (i, 0, 0)),
            pl.BlockSpec((1, 1, blk), lambda i: (i, 0, 0)),
            pl.BlockSpec((1, 1, blk), lambda i: (i, 0, 0)),
        ],
        out_specs=[
            pl.BlockSpec((blk, _D), lambda i: (i, 0)),
            pl.BlockSpec((1, 1, blk), lambda i: (i, 0, 0)),
        ],
        out_shape=[
            jax.ShapeDtypeStruct((_N, _D), jnp.float32),
            jax.ShapeDtypeStruct((nb, 1, blk), jnp.int32),
        ],
    )(emb, lab3, gid3, cat3)
    return en, pk3.reshape(_N)


_MESH = plsc.VectorSubcoreMesh(core_axis_name="c", subcore_axis_name="s")


def _bucket_idx(p, core, lane8):
    lab = p >> 4
    gid = (p >> 1) & 7
    cons = p & 1
    mylab = lab - core * _LAB_PER_SC
    mine = (cons == 1) & (mylab >= 0) & (mylab < _LAB_PER_SC)
    return jnp.where(mine, mylab * _NGRAPH + gid, _ROWS + lane8)


@functools.partial(
    pl.kernel,
    out_type=[
        jax.ShapeDtypeStruct((32, 32), jnp.float32),
        jax.ShapeDtypeStruct((32, _CT), jnp.float32),
    ],
    mesh=_MESH,
    compiler_params=pltpu.CompilerParams(needs_layout_passes=False),
    scratch_types=[
        pltpu.VMEM_SHARED((_TROWS, _D), jnp.float32),
        pltpu.VMEM((_CHUNK, _D), jnp.float32),
        pltpu.VMEM((_TAIL, _D), jnp.float32),
        pltpu.VMEM((_CT,), jnp.float32),
        pltpu.VMEM((_CHUNK,), jnp.int32),
        pltpu.VMEM((_TAIL,), jnp.int32),
        pltpu.VMEM((_CHUNK,), jnp.int32),
        pltpu.VMEM((_TAIL,), jnp.int32),
        pltpu.VMEM((8, _D), jnp.float32),
        pltpu.VMEM((16, _D), jnp.float32),
        pltpu.VMEM((2, 16), jnp.float32),
        pltpu.VMEM((32,), jnp.float32),
    ],
)
def _bucket_kernel(en_hbm, pk_hbm, out_hbm, cnt_hbm,
                   table, ebuf, ebuf_t, ctloc,
                   pbuf, pbuf_t, ibuf, ibuf_t,
                   zbuf, lbuf, abuf, obuf):
    c = lax.axis_index("c")
    s = lax.axis_index("s")
    lane = lax.iota(jnp.int32, 16)
    lane8 = lane & 7
    zv = jnp.zeros((16,), jnp.float32)
    onev = jnp.ones((16,), jnp.float32)

    for r in range(8):
        for q in range(_D // 16):
            zbuf[r, pl.ds(q * 16, 16)] = zv
    abuf[0, pl.ds(0, 16)] = zv
    abuf[1, pl.ds(0, 16)] = zv

    def _zero_ct(k, _):
        ctloc[pl.ds(k * 16, 16)] = zv
        return 0

    lax.fori_loop(0, _CT // 16, _zero_ct, 0)

    ngroups = _TROWS // 8

    def _zero_group(k, _):
        grp = k * 16 + s

        @pl.when(grp < ngroups)
        def _zt():
            pltpu.sync_copy(zbuf, table.at[pl.ds(grp * 8, 8), :])

        return 0

    lax.fori_loop(0, (ngroups + 15) // 16, _zero_group, 0)

    plsc.subcore_barrier()

    def _chunk_body(k, _):
        ch = k * 16 + s

        @pl.when(ch < _NFULL)
        def _full_chunk():
            n0 = ch * _CHUNK
            pltpu.sync_copy(pk_hbm.at[pl.ds(n0, _CHUNK)], pbuf)
            pltpu.sync_copy(en_hbm.at[pl.ds(n0, _CHUNK), :], ebuf)
            for j in range(_CHUNK // 16):
                p = pbuf[pl.ds(j * 16, 16)]
                idx = _bucket_idx(p, c, lane8)
                ibuf[pl.ds(j * 16, 16)] = idx
                plsc.addupdate_scatter(ctloc, [idx], onev)
            pltpu.sync_copy(ebuf, table.at[ibuf], add=True)

        return 0

    lax.fori_loop(0, 25, _chunk_body, 0)

    @pl.when(s == 6)
    def _tail_chunk():
        n0 = _NFULL * _CHUNK
        pltpu.sync_copy(pk_hbm.at[pl.ds(n0, _TAIL)], pbuf_t)
        pltpu.sync_copy(en_hbm.at[pl.ds(n0, _TAIL), :], ebuf_t)
        for j in range(_TAIL // 16):
            p = pbuf_t[pl.ds(j * 16, 16)]
            idx = _bucket_idx(p, c, lane8)
            ibuf_t[pl.ds(j * 16, 16)] = idx
            plsc.addupdate_scatter(ctloc, [idx], onev)
        pltpu.sync_copy(ebuf_t, table.at[ibuf_t], add=True)

    pltpu.sync_copy(ctloc, cnt_hbm.at[c * 16 + s])

    def _pair_body(j, _):
        lp = s * _PAIRS_PER_TILE + j

        @pl.when(lp < _LAB_PER_SC // 2)
        def _one_pair():
            pltpu.sync_copy(table.at[pl.ds(lp * 16, 16), :], lbuf)
            acc_a = jnp.zeros((16,), jnp.float32)
            acc_b = jnp.zeros((16,), jnp.float32)
            for half in range(2):
                svec = [jnp.zeros((16,), jnp.float32) for _ in range(_D // 16)]
                for g in range(_NGRAPH):
                    for q in range(_D // 16):
                        v = lbuf[half * 8 + g, pl.ds(q * 16, 16)]
                        acc_a = acc_a + v * v
                        svec[q] = svec[q] + v
                for q in range(_D // 16):
                    acc_b = acc_b + svec[q] * svec[q]
            abuf[0, pl.ds(0, 16)] = abuf[0, pl.ds(0, 16)] + acc_a
            abuf[1, pl.ds(0, 16)] = abuf[1, pl.ds(0, 16)] + acc_b

        return 0

    lax.fori_loop(0, _PAIRS_PER_TILE, _pair_body, 0)

    obuf[pl.ds(0, 16)] = abuf[0, pl.ds(0, 16)]
    obuf[pl.ds(16, 16)] = abuf[1, pl.ds(0, 16)]
    pltpu.sync_copy(obuf, out_hbm.at[c * 16 + s])


@functools.partial(
    pl.kernel,
    out_type=jax.ShapeDtypeStruct((32, 32), jnp.float32),
    mesh=_MESH,
    compiler_params=pltpu.CompilerParams(needs_layout_passes=False),
    scratch_types=[
        pltpu.VMEM((_N,), jnp.int32),
        pltpu.VMEM((_MAXSAMP,), jnp.int32),
        pltpu.VMEM((_MAXSAMP,), jnp.int32),
        pltpu.VMEM((16,), jnp.int32),
        pltpu.VMEM((64,), jnp.int32),
        pltpu.VMEM((64,), jnp.int32),
        pltpu.VMEM((64, _D), jnp.float32),
        pltpu.VMEM((64, _D), jnp.float32),
        pltpu.VMEM((256,), jnp.float32),
        pltpu.VMEM((32,), jnp.float32),
        pltpu.SemaphoreType.DMA,
        pltpu.SemaphoreType.DMA,
    ],
)
def _neg_kernel(en_hbm, pk_hbm, i1_hbm, i2_hbm, ns_hbm, out_hbm,
                pkbuf, i1buf, i2buf, nsbuf, g1buf, g2buf,
                rows1, rows2, tbuf, obuf, sem1, sem2):
    c = lax.axis_index("c")
    s = lax.axis_index("s")
    wid = c * 16 + s
    lane = lax.iota(jnp.int32, 16)
    pltpu.sync_copy(pk_hbm, pkbuf)
    pltpu.sync_copy(i1_hbm, i1buf)
    pltpu.sync_copy(i2_hbm, i2buf)
    pltpu.sync_copy(ns_hbm, nsbuf)
    nsamp = nsbuf[pl.ds(0, 16)][0]
    obuf[pl.ds(0, 16)] = jnp.zeros((16,), jnp.float32)
    obuf[pl.ds(16, 16)] = jnp.zeros((16,), jnp.float32)

    def _do_subgroup(base, u, rows1_off, validf):
        acc0 = jnp.zeros((16,), jnp.float32)
        for j in range(16):
            r = rows1_off + j
            acc = rows1[r, pl.ds(0, 16)] * rows2[r, pl.ds(0, 16)]
            for q in range(1, _D // 16):
                acc = acc + (rows1[r, pl.ds(q * 16, 16)]
                             * rows2[r, pl.ds(q * 16, 16)])
            tbuf[pl.ds(j * 16, 16)] = acc
        dots = plsc.load_gather(tbuf, [lane * 16])
        for q in range(1, 16):
            dots = dots + plsc.load_gather(tbuf, [lane * 16 + q])
        dv = jnp.maximum(dots, 0.0) * validf
        obuf[pl.ds(0, 16)] = obuf[pl.ds(0, 16)] + dv
        obuf[pl.ds(16, 16)] = obuf[pl.ds(16, 16)] + validf

    def _valid_mask(i1, i2, sid0):
        p1 = plsc.load_gather(pkbuf, [i1])
        p2 = plsc.load_gather(pkbuf, [i2])
        valid = (((p1 >> 4) != (p2 >> 4))
                 & (((p1 >> 1) & 7) != ((p2 >> 1) & 7))
                 & (((p1 | p2) & 1) == 1)
                 & ((sid0 + lane) < nsamp))
        return jnp.where(valid, 1.0, 0.0).astype(jnp.float32)

    nfull = _MAXSAMP // 64

    def _chunk(k, _):
        ch = k * 32 + wid

        @pl.when(ch < nfull)
        def _full():
            base = ch * 64
            for u in range(4):
                g1buf[pl.ds(u * 16, 16)] = i1buf[pl.ds(base + u * 16, 16)]
                g2buf[pl.ds(u * 16, 16)] = i2buf[pl.ds(base + u * 16, 16)]
            d1 = pltpu.async_copy(en_hbm.at[g1buf], rows1, sem1)
            d2 = pltpu.async_copy(en_hbm.at[g2buf], rows2, sem2)
            vfs = []
            for u in range(4):
                vfs.append(_valid_mask(g1buf[pl.ds(u * 16, 16)],
                                       g2buf[pl.ds(u * 16, 16)],
                                       base + u * 16))
            d1.wait()
            d2.wait()
            for u in range(4):
                _do_subgroup(base, u, u * 16, vfs[u])

        return 0

    lax.fori_loop(0, (nfull + 31) // 32, _chunk, 0)

    @pl.when(wid == 28)
    def _tail():
        base = nfull * 64
        g1buf[pl.ds(0, 16)] = i1buf[pl.ds(base, 16)]
        g2buf[pl.ds(0, 16)] = i2buf[pl.ds(base, 16)]
        d1 = pltpu.async_copy(
            en_hbm.at[g1buf.at[pl.ds(0, 16)]], rows1.at[pl.ds(0, 16), :], sem1)
        d2 = pltpu.async_copy(
            en_hbm.at[g2buf.at[pl.ds(0, 16)]], rows2.at[pl.ds(0, 16), :], sem2)
        vf = _valid_mask(g1buf[pl.ds(0, 16)], g2buf[pl.ds(0, 16)], base)
        d1.wait()
        d2.wait()
        _do_subgroup(base, 0, 0, vf)

    pltpu.sync_copy(obuf, out_hbm.at[wid])


def _neg_indices(n_pairs, n_total):
    nk = jax.random.key(123)
    nk1, nk2 = jax.random.split(nk)
    bidx = jnp.clip(n_pairs, 0, 200)
    n_samples = jnp.where(bidx < 200, bidx * 50, _MAXSAMP)

    def make_branch(sz):
        def branch(keys):
            ka, kb = keys
            i1 = jax.random.randint(ka, (sz,), 0, n_total)
            i2 = jax.random.randint(kb, (sz,), 0, n_total)
            return (jnp.pad(i1, (0, _MAXSAMP - sz)),
                    jnp.pad(i2, (0, _MAXSAMP - sz)))
        return branch

    branches = [make_branch(50 * t) for t in range(200)] + [make_branch(_MAXSAMP)]
    idx1, idx2 = lax.switch(bidx, branches, (nk1, nk2))
    return idx1.astype(jnp.int32), idx2.astype(jnp.int32), n_samples


def kernel(embeddings, labels, graph_ids, categories):
    en, packed = _norm_pack(embeddings, labels.astype(jnp.int32),
                            graph_ids.astype(jnp.int32),
                            categories.astype(jnp.int32))
    nk = jax.random.key(123)
    nk1, nk2 = jax.random.split(nk)
    idx1_c = jax.random.randint(nk1, (_MAXSAMP,), 0, _N).astype(jnp.int32)
    idx2_c = jax.random.randint(nk2, (_MAXSAMP,), 0, _N).astype(jnp.int32)
    ns_full = jnp.full((16,), _MAXSAMP, dtype=jnp.int32)
    negparts_spec = _neg_kernel(en, packed, idx1_c, idx2_c, ns_full)
    parts, cnts = _bucket_kernel(en, packed)
    a = jnp.sum(parts[:, :16])
    b = jnp.sum(parts[:, 16:])
    csum = cnts.reshape(2, 16, _CT).sum(axis=1)
    n_lg = csum[:, :_ROWS].reshape(_NLAB, _NGRAPH)
    cq = jnp.sum(n_lg * n_lg)
    n_l = jnp.sum(n_lg, axis=-1)
    dq = jnp.sum(n_l * n_l)
    n_pairs_f = 0.5 * (dq - cq)
    pos_sim = 0.5 * (b - a)
    pos_sum = n_pairs_f - pos_sim
    n_pairs = jnp.round(n_pairs_f).astype(jnp.int32)
    pos_loss = pos_sum / jnp.maximum(n_pairs, 1).astype(jnp.float32)

    def _rare_path(_):
        idx1, idx2, n_samples = _neg_indices(n_pairs, _N)
        ns_arr = jnp.full((16,), n_samples, dtype=jnp.int32)
        return _neg_kernel(en, packed, idx1, idx2, ns_arr)

    negparts = lax.cond(n_pairs >= 200,
                        lambda _: negparts_spec, _rare_path, 0)
    nsum = jnp.sum(negparts[:, :16])
    n_valid = jnp.round(jnp.sum(negparts[:, 16:])).astype(jnp.int32)
    neg_loss = nsum / jnp.maximum(n_valid, 1).astype(jnp.float32)
    total = jnp.where(n_valid > 0, pos_loss + neg_loss, pos_loss)
    return jnp.where(n_pairs > 0, total, jnp.float32(0.0))


# speedup vs baseline: 121.5372x; 1.2302x over previous
import functools

import jax
import jax.numpy as jnp
from jax import lax
from jax.experimental import pallas as pl
from jax.experimental.pallas import tpu as pltpu
from jax.experimental.pallas import tpu_sc as plsc

_N = 50000
_D = 128
_NLAB = 2000
_NGRAPH = 8
_LAB_PER_SC = _NLAB // 2
_ROWS = _LAB_PER_SC * _NGRAPH
_NDUMMY = 8
_TROWS = _ROWS + _NDUMMY
_NBUCKET = _NLAB * _NGRAPH
_CT = 8192
_CHUNK = 128
_NFULL = _N // _CHUNK
_TAIL = _N - _NFULL * _CHUNK
_PAIRS_PER_TILE = 32
_MAXSAMP = 10000
_NEG_GROUPS = _MAXSAMP // 16


def _norm_pack_body(e_ref, lab_ref, gid_ref, cat_ref, en_ref, pk_ref):
    e = e_ref[...]
    nrm = jnp.clip(jnp.sqrt(jnp.sum(e * e, axis=1, keepdims=True)), 1e-12)
    en_ref[...] = e / nrm
    lab = lab_ref[...]
    gid = gid_ref[...]
    cat = cat_ref[...]
    pk_ref[...] = lab * 16 + gid * 2 + (cat < 3).astype(jnp.int32)


def _norm_pack(emb, labels, gids, cats):
    nb = 25
    blk = _N // nb
    lab3 = labels.reshape(nb, 1, blk)
    gid3 = gids.reshape(nb, 1, blk)
    cat3 = cats.reshape(nb, 1, blk)
    en, pk3 = pl.pallas_call(
        _norm_pack_body,
        grid=(nb,),
        in_specs=[
            pl.BlockSpec((blk, _D), lambda i: (i, 0)),
            pl.BlockSpec((1, 1, blk), lambda i: (i, 0, 0)),
            pl.BlockSpec((1, 1, blk), lambda i: (i, 0, 0)),
            pl.BlockSpec((1, 1, blk), lambda i: (i, 0, 0)),
        ],
        out_specs=[
            pl.BlockSpec((blk, _D), lambda i: (i, 0)),
            pl.BlockSpec((1, 1, blk), lambda i: (i, 0, 0)),
        ],
        out_shape=[
            jax.ShapeDtypeStruct((_N, _D), jnp.float32),
            jax.ShapeDtypeStruct((nb, 1, blk), jnp.int32),
        ],
    )(emb, lab3, gid3, cat3)
    return en, pk3.reshape(_N)


_MESH = plsc.VectorSubcoreMesh(core_axis_name="c", subcore_axis_name="s")


def _bucket_idx(p, core, lane8):
    lab = p >> 4
    gid = (p >> 1) & 7
    cons = p & 1
    mylab = lab - core * _LAB_PER_SC
    mine = (cons == 1) & (mylab >= 0) & (mylab < _LAB_PER_SC)
    return jnp.where(mine, mylab * _NGRAPH + gid, _ROWS + lane8)


@functools.partial(
    pl.kernel,
    out_type=[
        jax.ShapeDtypeStruct((32, 32), jnp.float32),
        jax.ShapeDtypeStruct((32, _CT), jnp.float32),
    ],
    mesh=_MESH,
    compiler_params=pltpu.CompilerParams(needs_layout_passes=False),
    scratch_types=[
        pltpu.VMEM_SHARED((_TROWS, _D), jnp.float32),
        pltpu.VMEM((_CHUNK, _D), jnp.float32),
        pltpu.VMEM((_CHUNK, _D), jnp.float32),
        pltpu.VMEM((_TAIL, _D), jnp.float32),
        pltpu.VMEM((_CT,), jnp.float32),
        pltpu.VMEM((_CHUNK,), jnp.int32),
        pltpu.VMEM((_CHUNK,), jnp.int32),
        pltpu.VMEM((_TAIL,), jnp.int32),
        pltpu.VMEM((_CHUNK,), jnp.int32),
        pltpu.VMEM((_CHUNK,), jnp.int32),
        pltpu.VMEM((_TAIL,), jnp.int32),
        pltpu.VMEM((8, _D), jnp.float32),
        pltpu.VMEM((16, _D), jnp.float32),
        pltpu.VMEM((2, 16), jnp.float32),
        pltpu.VMEM((32,), jnp.float32),
        pltpu.SemaphoreType.DMA,
        pltpu.SemaphoreType.DMA,
        pltpu.SemaphoreType.DMA,
        pltpu.SemaphoreType.DMA,
    ],
)
def _bucket_kernel(en_hbm, pk_hbm, out_hbm, cnt_hbm,
                   table, ebuf, ebuf2, ebuf_t, ctloc,
                   pbuf, pbuf2, pbuf_t, ibuf, ibuf2, ibuf_t,
                   zbuf, lbuf, abuf, obuf,
                   sem_pa, sem_ea, sem_pb, sem_eb):
    c = lax.axis_index("c")
    s = lax.axis_index("s")
    lane = lax.iota(jnp.int32, 16)
    lane8 = lane & 7
    zv = jnp.zeros((16,), jnp.float32)
    onev = jnp.ones((16,), jnp.float32)

    for r in range(8):
        for q in range(_D // 16):
            zbuf[r, pl.ds(q * 16, 16)] = zv
    abuf[0, pl.ds(0, 16)] = zv
    abuf[1, pl.ds(0, 16)] = zv

    def _zero_ct(k, _):
        ctloc[pl.ds(k * 16, 16)] = zv
        return 0

    lax.fori_loop(0, _CT // 16, _zero_ct, 0)

    ngroups = _TROWS // 8

    def _zero_group(k, _):
        grp = k * 16 + s

        @pl.when(grp < ngroups)
        def _zt():
            pltpu.sync_copy(zbuf, table.at[pl.ds(grp * 8, 8), :])

        return 0

    lax.fori_loop(0, (ngroups + 15) // 16, _zero_group, 0)

    plsc.subcore_barrier()

    def _fire(k, pb, eb, sp, se):
        n0 = (k * 16 + s) * _CHUNK
        pltpu.async_copy(pk_hbm.at[pl.ds(n0, _CHUNK)], pb, sp)
        pltpu.async_copy(en_hbm.at[pl.ds(n0, _CHUNK), :], eb, se)

    def _drain(k, pb, eb, sp, se):
        n0 = (k * 16 + s) * _CHUNK
        pltpu.make_async_copy(pk_hbm.at[pl.ds(n0, _CHUNK)], pb, sp).wait()
        pltpu.make_async_copy(en_hbm.at[pl.ds(n0, _CHUNK), :], eb, se).wait()

    def _process(pb, eb, ib):
        for j in range(_CHUNK // 16):
            p = pb[pl.ds(j * 16, 16)]
            idx = _bucket_idx(p, c, lane8)
            ib[pl.ds(j * 16, 16)] = idx
            plsc.addupdate_scatter(ctloc, [idx], onev)
        pltpu.sync_copy(eb, table.at[ib], add=True)

    _fire(0, pbuf, ebuf, sem_pa, sem_ea)

    def _chunk_body(t, _):
        k0 = 2 * t
        k1 = 2 * t + 1

        @pl.when(k1 * 16 + s < _NFULL)
        def _fire_b():
            _fire(k1, pbuf2, ebuf2, sem_pb, sem_eb)

        @pl.when(k0 * 16 + s < _NFULL)
        def _do_a():
            _drain(k0, pbuf, ebuf, sem_pa, sem_ea)
            _process(pbuf, ebuf, ibuf)

        @pl.when((k0 + 2) * 16 + s < _NFULL)
        def _fire_a():
            _fire(k0 + 2, pbuf, ebuf, sem_pa, sem_ea)

        @pl.when(k1 * 16 + s < _NFULL)
        def _do_b():
            _drain(k1, pbuf2, ebuf2, sem_pb, sem_eb)
            _process(pbuf2, ebuf2, ibuf2)

        return 0

    lax.fori_loop(0, 13, _chunk_body, 0)

    @pl.when(s == 6)
    def _tail_chunk():
        n0 = _NFULL * _CHUNK
        pltpu.sync_copy(pk_hbm.at[pl.ds(n0, _TAIL)], pbuf_t)
        pltpu.sync_copy(en_hbm.at[pl.ds(n0, _TAIL), :], ebuf_t)
        for j in range(_TAIL // 16):
            p = pbuf_t[pl.ds(j * 16, 16)]
            idx = _bucket_idx(p, c, lane8)
            ibuf_t[pl.ds(j * 16, 16)] = idx
            plsc.addupdate_scatter(ctloc, [idx], onev)
        pltpu.sync_copy(ebuf_t, table.at[ibuf_t], add=True)

    pltpu.sync_copy(ctloc, cnt_hbm.at[c * 16 + s])

    def _pair_body(j, _):
        lp = s * _PAIRS_PER_TILE + j

        @pl.when(lp < _LAB_PER_SC // 2)
        def _one_pair():
            pltpu.sync_copy(table.at[pl.ds(lp * 16, 16), :], lbuf)
            acc_a = jnp.zeros((16,), jnp.float32)
            acc_b = jnp.zeros((16,), jnp.float32)
            for half in range(2):
                svec = [jnp.zeros((16,), jnp.float32) for _ in range(_D // 16)]
                for g in range(_NGRAPH):
                    for q in range(_D // 16):
                        v = lbuf[half * 8 + g, pl.ds(q * 16, 16)]
                        acc_a = acc_a + v * v
                        svec[q] = svec[q] + v
                for q in range(_D // 16):
                    acc_b = acc_b + svec[q] * svec[q]
            abuf[0, pl.ds(0, 16)] = abuf[0, pl.ds(0, 16)] + acc_a
            abuf[1, pl.ds(0, 16)] = abuf[1, pl.ds(0, 16)] + acc_b

        return 0

    lax.fori_loop(0, _PAIRS_PER_TILE, _pair_body, 0)

    obuf[pl.ds(0, 16)] = abuf[0, pl.ds(0, 16)]
    obuf[pl.ds(16, 16)] = abuf[1, pl.ds(0, 16)]
    pltpu.sync_copy(obuf, out_hbm.at[c * 16 + s])


@functools.partial(
    pl.kernel,
    out_type=jax.ShapeDtypeStruct((32, 32), jnp.float32),
    mesh=_MESH,
    compiler_params=pltpu.CompilerParams(needs_layout_passes=False),
    scratch_types=[
        pltpu.VMEM((_N,), jnp.int32),
        pltpu.VMEM((_MAXSAMP,), jnp.int32),
        pltpu.VMEM((_MAXSAMP,), jnp.int32),
        pltpu.VMEM((16,), jnp.int32),
        pltpu.VMEM((64,), jnp.int32),
        pltpu.VMEM((64,), jnp.int32),
        pltpu.VMEM((64, _D), jnp.float32),
        pltpu.VMEM((64, _D), jnp.float32),
        pltpu.VMEM((256,), jnp.float32),
        pltpu.VMEM((32,), jnp.float32),
        pltpu.SemaphoreType.DMA,
        pltpu.SemaphoreType.DMA,
    ],
)
def _neg_kernel(en_hbm, pk_hbm, i1_hbm, i2_hbm, ns_hbm, out_hbm,
                pkbuf, i1buf, i2buf, nsbuf, g1buf, g2buf,
                rows1, rows2, tbuf, obuf, sem1, sem2):
    c = lax.axis_index("c")
    s = lax.axis_index("s")
    wid = c * 16 + s
    lane = lax.iota(jnp.int32, 16)
    pltpu.sync_copy(pk_hbm, pkbuf)
    pltpu.sync_copy(i1_hbm, i1buf)
    pltpu.sync_copy(i2_hbm, i2buf)
    pltpu.sync_copy(ns_hbm, nsbuf)
    nsamp = nsbuf[pl.ds(0, 16)][0]
    obuf[pl.ds(0, 16)] = jnp.zeros((16,), jnp.float32)
    obuf[pl.ds(16, 16)] = jnp.zeros((16,), jnp.float32)

    def _do_subgroup(base, u, rows1_off, validf):
        acc0 = jnp.zeros((16,), jnp.float32)
        for j in range(16):
            r = rows1_off + j
            acc = rows1[r, pl.ds(0, 16)] * rows2[r, pl.ds(0, 16)]
            for q in range(1, _D // 16):
                acc = acc + (rows1[r, pl.ds(q * 16, 16)]
                             * rows2[r, pl.ds(q * 16, 16)])
            tbuf[pl.ds(j * 16, 16)] = acc
        dots = plsc.load_gather(tbuf, [lane * 16])
        for q in range(1, 16):
            dots = dots + plsc.load_gather(tbuf, [lane * 16 + q])
        dv = jnp.maximum(dots, 0.0) * validf
        obuf[pl.ds(0, 16)] = obuf[pl.ds(0, 16)] + dv
        obuf[pl.ds(16, 16)] = obuf[pl.ds(16, 16)] + validf

    def _valid_mask(i1, i2, sid0):
        p1 = plsc.load_gather(pkbuf, [i1])
        p2 = plsc.load_gather(pkbuf, [i2])
        valid = (((p1 >> 4) != (p2 >> 4))
                 & (((p1 >> 1) & 7) != ((p2 >> 1) & 7))
                 & (((p1 | p2) & 1) == 1)
                 & ((sid0 + lane) < nsamp))
        return jnp.where(valid, 1.0, 0.0).astype(jnp.float32)

    nfull = _MAXSAMP // 64

    def _chunk(k, _):
        ch = k * 32 + wid

        @pl.when(ch < nfull)
        def _full():
            base = ch * 64
            for u in range(4):
                g1buf[pl.ds(u * 16, 16)] = i1buf[pl.ds(base + u * 16, 16)]
                g2buf[pl.ds(u * 16, 16)] = i2buf[pl.ds(base + u * 16, 16)]
            d1 = pltpu.async_copy(en_hbm.at[g1buf], rows1, sem1)
            d2 = pltpu.async_copy(en_hbm.at[g2buf], rows2, sem2)
            vfs = []
            for u in range(4):
                vfs.append(_valid_mask(g1buf[pl.ds(u * 16, 16)],
                                       g2buf[pl.ds(u * 16, 16)],
                                       base + u * 16))
            d1.wait()
            d2.wait()
            for u in range(4):
                _do_subgroup(base, u, u * 16, vfs[u])

        return 0

    lax.fori_loop(0, (nfull + 31) // 32, _chunk, 0)

    @pl.when(wid == 28)
    def _tail():
        base = nfull * 64
        g1buf[pl.ds(0, 16)] = i1buf[pl.ds(base, 16)]
        g2buf[pl.ds(0, 16)] = i2buf[pl.ds(base, 16)]
        d1 = pltpu.async_copy(
            en_hbm.at[g1buf.at[pl.ds(0, 16)]], rows1.at[pl.ds(0, 16), :], sem1)
        d2 = pltpu.async_copy(
            en_hbm.at[g2buf.at[pl.ds(0, 16)]], rows2.at[pl.ds(0, 16), :], sem2)
        vf = _valid_mask(g1buf[pl.ds(0, 16)], g2buf[pl.ds(0, 16)], base)
        d1.wait()
        d2.wait()
        _do_subgroup(base, 0, 0, vf)

    pltpu.sync_copy(obuf, out_hbm.at[wid])


def _neg_indices(n_pairs, n_total):
    nk = jax.random.key(123)
    nk1, nk2 = jax.random.split(nk)
    bidx = jnp.clip(n_pairs, 0, 200)
    n_samples = jnp.where(bidx < 200, bidx * 50, _MAXSAMP)

    def make_branch(sz):
        def branch(keys):
            ka, kb = keys
            i1 = jax.random.randint(ka, (sz,), 0, n_total)
            i2 = jax.random.randint(kb, (sz,), 0, n_total)
            return (jnp.pad(i1, (0, _MAXSAMP - sz)),
                    jnp.pad(i2, (0, _MAXSAMP - sz)))
        return branch

    branches = [make_branch(50 * t) for t in range(200)] + [make_branch(_MAXSAMP)]
    idx1, idx2 = lax.switch(bidx, branches, (nk1, nk2))
    return idx1.astype(jnp.int32), idx2.astype(jnp.int32), n_samples


def kernel(embeddings, labels, graph_ids, categories):
    en, packed = _norm_pack(embeddings, labels.astype(jnp.int32),
                            graph_ids.astype(jnp.int32),
                            categories.astype(jnp.int32))
    nk = jax.random.key(123)
    nk1, nk2 = jax.random.split(nk)
    idx1_c = jax.random.randint(nk1, (_MAXSAMP,), 0, _N).astype(jnp.int32)
    idx2_c = jax.random.randint(nk2, (_MAXSAMP,), 0, _N).astype(jnp.int32)
    ns_full = jnp.full((16,), _MAXSAMP, dtype=jnp.int32)
    negparts_spec = _neg_kernel(en, packed, idx1_c, idx2_c, ns_full)
    parts, cnts = _bucket_kernel(en, packed)
    a = jnp.sum(parts[:, :16])
    b = jnp.sum(parts[:, 16:])
    csum = cnts.reshape(2, 16, _CT).sum(axis=1)
    n_lg = csum[:, :_ROWS].reshape(_NLAB, _NGRAPH)
    cq = jnp.sum(n_lg * n_lg)
    n_l = jnp.sum(n_lg, axis=-1)
    dq = jnp.sum(n_l * n_l)
    n_pairs_f = 0.5 * (dq - cq)
    pos_sim = 0.5 * (b - a)
    pos_sum = n_pairs_f - pos_sim
    n_pairs = jnp.round(n_pairs_f).astype(jnp.int32)
    pos_loss = pos_sum / jnp.maximum(n_pairs, 1).astype(jnp.float32)

    def _rare_path(_):
        idx1, idx2, n_samples = _neg_indices(n_pairs, _N)
        ns_arr = jnp.full((16,), n_samples, dtype=jnp.int32)
        return _neg_kernel(en, packed, idx1, idx2, ns_arr)

    negparts = lax.cond(n_pairs >= 200,
                        lambda _: negparts_spec, _rare_path, 0)
    nsum = jnp.sum(negparts[:, :16])
    n_valid = jnp.round(jnp.sum(negparts[:, 16:])).astype(jnp.int32)
    neg_loss = nsum / jnp.maximum(n_valid, 1).astype(jnp.float32)
    total = jnp.where(n_valid > 0, pos_loss + neg_loss, pos_loss)
    return jnp.where(n_pairs > 0, total, jnp.float32(0.0))
